# Initial kernel scaffold; baseline (speedup 1.0000x reference)
import jax
import jax.numpy as jnp
from jax.experimental import pallas as pl


def kernel(interactions, emb_w, fc_w, bias):
    raise NotImplementedError("write your pallas kernel here")


# speedup vs baseline: 1.3158x; 1.3158x over previous
import jax
import jax.numpy as jnp
from jax import lax
from jax.experimental import pallas as pl
from jax.experimental.pallas import tpu as pltpu
from jax.experimental.pallas import tpu_sc as plsc

B = 16384
F = 26
D = 16
NC = 2
NS = 16
NW = NC * NS
BW = B // NW
IDXW = BW * F
IPR = 128
NROW = IDXW // IPR
CB = 64
NCHUNK = BW // CB
RPC = CB * F // IPR
LPC = CB * F


def _fm_body(idx_hbm, emb_hbm, fc_hbm, bias_hbm, out_hbm,
             idx_v, emb_v, fc_v, t_v, out_v, bias_v, sem):
    wid = lax.axis_index("s") * NC + lax.axis_index("c")
    ibase = wid * NROW

    pltpu.sync_copy(bias_hbm, bias_v)
    pltpu.sync_copy(idx_hbm.at[pl.ds(ibase, NROW)], idx_v)

    for c in range(NCHUNK):
        copies = []
        for j in range(RPC):
            r = c * RPC + j
            copies.append(pltpu.async_copy(
                emb_hbm.at[idx_v.at[r]],
                emb_v.at[pl.ds(j * IPR, IPR)], sem))
            copies.append(pltpu.async_copy(
                fc_hbm.at[idx_v.at[r]],
                fc_v.at[pl.ds(r * IPR, IPR)], sem))
        for cp in copies:
            cp.wait()

        def row_body(rr, carry):
            base = rr * F
            v0 = emb_v[base, :]
            s = v0
            ssq = v0 * v0
            for f in range(1, F):
                v = emb_v[base + f, :]
                s = s + v
                ssq = ssq + v * v
            t_v[pl.ds((c * CB + rr) * D, D)] = s * s - ssq
            return carry

        lax.fori_loop(0, CB, row_body, 0, unroll=2)

    lane = lax.iota(jnp.int32, 16)
    bias_vec = bias_v[:]

    def grp_body(g, carry):
        rows = g * 16 + lane
        trows = rows * D
        acc_t = plsc.load_gather(t_v, [trows])
        for d in range(1, D):
            acc_t = acc_t + plsc.load_gather(t_v, [trows + d])
        frows = rows * F
        acc_f = plsc.load_gather(fc_v, [frows])
        for f in range(1, F):
            acc_f = acc_f + plsc.load_gather(fc_v, [frows + f])
        out_v[pl.ds(g * 16, 16)] = acc_f + bias_vec + 0.5 * acc_t
        return carry

    lax.fori_loop(0, BW // 16, grp_body, 0)

    pltpu.sync_copy(out_v, out_hbm.at[pl.ds(wid * BW, BW)])


def kernel(interactions, emb_w, fc_w, bias):
    idx = interactions.reshape(NW * NROW, IPR)
    bias16 = jnp.broadcast_to(bias, (16,))
    mesh = plsc.VectorSubcoreMesh(core_axis_name="c", subcore_axis_name="s")
    fm = pl.kernel(
        _fm_body,
        out_type=jax.ShapeDtypeStruct((B,), jnp.float32),
        mesh=mesh,
        compiler_params=pltpu.CompilerParams(
            needs_layout_passes=False, use_tc_tiling_on_sc=False),
        scratch_types=[
            pltpu.VMEM((NROW, IPR), jnp.int32),
            pltpu.VMEM((LPC, D), jnp.float32),
            pltpu.VMEM((IDXW,), jnp.float32),
            pltpu.VMEM((BW * D,), jnp.float32),
            pltpu.VMEM((BW,), jnp.float32),
            pltpu.VMEM((16,), jnp.float32),
            pltpu.SemaphoreType.DMA,
        ],
    )
    return fm(idx, emb_w, fc_w.reshape(-1), bias16)


# speedup vs baseline: 1.3573x; 1.0316x over previous
import jax
import jax.numpy as jnp
from jax import lax
from jax.experimental import pallas as pl
from jax.experimental.pallas import tpu as pltpu
from jax.experimental.pallas import tpu_sc as plsc

B = 16384
F = 26
D = 16
NC = 2
NS = 16
NW = NC * NS
BW = B // NW
IDXW = BW * F
IPR = 128
NROW = IDXW // IPR
CB = 64
NCHUNK = BW // CB
RPC = CB * F // IPR
LPC = CB * F


def _fm_body(idx_hbm, emb_hbm, fc_hbm, bias_hbm, out_hbm,
             idx_v, emb_v, fc_v, t_v, out_v, bias_v, sem, fsem):
    wid = lax.axis_index("s") * NC + lax.axis_index("c")
    ibase = wid * NROW

    pltpu.sync_copy(bias_hbm, bias_v)
    pltpu.sync_copy(idx_hbm.at[pl.ds(ibase, NROW)], idx_v)

    def fire_emb(c):
        buf = c % 2
        cps = []
        for j in range(RPC):
            r = c * RPC + j
            cps.append(pltpu.async_copy(
                emb_hbm.at[idx_v.at[r]],
                emb_v.at[pl.ds((buf * RPC + j) * IPR, IPR)], sem))
        return cps

    fc_copies = [pltpu.async_copy(
        fc_hbm.at[idx_v.at[r]],
        fc_v.at[pl.ds(r * IPR, IPR)], fsem) for r in range(NROW)]

    pend = fire_emb(0)
    for c in range(NCHUNK):
        for cp in pend:
            cp.wait()
        if c + 1 < NCHUNK:
            pend = fire_emb(c + 1)
        base0 = (c % 2) * LPC

        def row_body(rr, carry):
            base = base0 + rr * F
            v0 = emb_v[base, :]
            s = v0
            ssq = v0 * v0
            for f in range(1, F):
                v = emb_v[base + f, :]
                s = s + v
                ssq = ssq + v * v
            t_v[pl.ds((c * CB + rr) * D, D)] = s * s - ssq
            return carry

        lax.fori_loop(0, CB, row_body, 0, unroll=2)

    for cp in fc_copies:
        cp.wait()

    lane = lax.iota(jnp.int32, 16)
    bias_vec = bias_v[:]

    def grp_body(g, carry):
        rows = g * 16 + lane
        trows = rows * D
        acc_t = plsc.load_gather(t_v, [trows])
        for d in range(1, D):
            acc_t = acc_t + plsc.load_gather(t_v, [trows + d])
        frows = rows * F
        acc_f = plsc.load_gather(fc_v, [frows])
        for f in range(1, F):
            acc_f = acc_f + plsc.load_gather(fc_v, [frows + f])
        out_v[pl.ds(g * 16, 16)] = acc_f + bias_vec + 0.5 * acc_t
        return carry

    lax.fori_loop(0, BW // 16, grp_body, 0)

    pltpu.sync_copy(out_v, out_hbm.at[pl.ds(wid * BW, BW)])


def kernel(interactions, emb_w, fc_w, bias):
    idx = interactions.reshape(NW * NROW, IPR)
    bias16 = jnp.broadcast_to(bias, (16,))
    mesh = plsc.VectorSubcoreMesh(core_axis_name="c", subcore_axis_name="s")
    fm = pl.kernel(
        _fm_body,
        out_type=jax.ShapeDtypeStruct((B,), jnp.float32),
        mesh=mesh,
        compiler_params=pltpu.CompilerParams(
            needs_layout_passes=False, use_tc_tiling_on_sc=False),
        scratch_types=[
            pltpu.VMEM((NROW, IPR), jnp.int32),
            pltpu.VMEM((2 * LPC, D), jnp.float32),
            pltpu.VMEM((IDXW,), jnp.float32),
            pltpu.VMEM((BW * D,), jnp.float32),
            pltpu.VMEM((BW,), jnp.float32),
            pltpu.VMEM((16,), jnp.float32),
            pltpu.SemaphoreType.DMA,
            pltpu.SemaphoreType.DMA,
        ],
    )
    return fm(idx, emb_w, fc_w.reshape(-1), bias16)
